# baseline (speedup 1.0000x reference)
import functools

import jax
import jax.numpy as jnp
import numpy as np
from jax import lax
from jax.experimental import pallas as pl
from jax.experimental.pallas import tpu as pltpu
from jax.experimental.pallas import tpu_sc as plsc

_GRID = 0.5
_SCALE_MULT = 3.0
_PC_MIN = np.array([-50.0, -50.0, -5.0], dtype=np.float32)

_NCELL_X = 8
_NCELL_Y = 8
_NCELLS = _NCELL_X * _NCELL_Y
_CELL_VOX = 25
_LCAP = 192
_NG = 1024
_NPTS = 8192
_NCLS = 17
_NW = 32
_PPW = _NPTS // _NW
_OVF_BASE = _NCELLS * _LCAP
_TRASH = _NCELLS * _LCAP + 4 * _NG
_LIST_LEN = _TRASH + 16
_NPF = 8
_NGR = 24


def _prep_kernel(pts_ref, mm_ref, sc_ref, cd_ref, op_ref, gout_ref, pout_ref):
    pcm = [-50.0, -50.0, -5.0]
    for r in range(3):
        x = pts_ref[r]
        ti = ((x - pcm[r]) / _GRID).astype(jnp.int32).astype(jnp.float32)
        pout_ref[:, r * _PPW:(r + 1) * _PPW] = x
        pout_ref[:, (3 + r) * _PPW:(4 + r) * _PPW] = ti
        if r == 0:
            cx = jnp.clip(jnp.floor(ti / _CELL_VOX), 0, _NCELL_X - 1)
        elif r == 1:
            cy = jnp.clip(jnp.floor(ti / _CELL_VOX), 0, _NCELL_Y - 1)
    pout_ref[:, 6 * _PPW:7 * _PPW] = cx * _NCELL_Y + cy
    pout_ref[:, 7 * _PPW:8 * _PPW] = jnp.zeros((_NW, _PPW), jnp.float32)

    mm = mm_ref[...]
    sc = sc_ref[...]
    cd = cd_ref[...]
    op = op_ref[...]
    mint = []
    for r in range(3):
        mi = ((mm[r:r + 1, :] - pcm[r]) / _GRID).astype(jnp.int32)
        mint.append(mi.astype(jnp.float32))
        gout_ref[r:r + 1, :] = mm[r:r + 1, :]
        gout_ref[3 + r:4 + r, :] = mint[r]
        gout_ref[7 + r:8 + r, :] = -0.5 * cd[r:r + 1, :]
    radii = jnp.ceil(jnp.maximum(jnp.maximum(sc[0:1, :], sc[1:2, :]),
                                 sc[2:3, :]) * _SCALE_MULT / _GRID)
    gout_ref[6:7, :] = radii
    gout_ref[10:11, :] = jnp.log(op)

    x0 = jnp.clip(jnp.floor((mint[0] - radii) / _CELL_VOX), 0, _NCELL_X - 1)
    x1 = jnp.clip(jnp.floor((mint[0] + radii) / _CELL_VOX), 0, _NCELL_X - 1)
    y0 = jnp.clip(jnp.floor((mint[1] - radii) / _CELL_VOX), 0, _NCELL_Y - 1)
    y1 = jnp.clip(jnp.floor((mint[1] + radii) / _CELL_VOX), 0, _NCELL_Y - 1)
    s0 = x0 * _NCELL_Y + y0
    s1 = x0 * _NCELL_Y + y1
    s2 = x1 * _NCELL_Y + y0
    s3 = x1 * _NCELL_Y + y1
    s1 = jnp.where(s1 == s0, -1.0, s1)
    s2 = jnp.where(s2 == s0, -1.0, s2)
    s3 = jnp.where((s3 == s0) | (s3 == s1) | (s3 == s2), -1.0, s3)
    slots = [s0, s1, s2, s3]

    cells_iota = lax.broadcasted_iota(jnp.int32, (_NG, _NCELLS), 1
                                      ).astype(jnp.float32)
    og = jnp.zeros((_NG, _NCELLS), jnp.float32)
    for s in slots:
        og = og + (s.reshape(_NG, 1) == cells_iota).astype(jnp.float32)

    ii = lax.broadcasted_iota(jnp.int32, (_NG, _NG), 0)
    jj = lax.broadcasted_iota(jnp.int32, (_NG, _NG), 1)
    tril = (ii > jj).astype(jnp.float32)
    prev = jnp.dot(tril, og, preferred_element_type=jnp.float32)
    counts = jnp.minimum(jnp.sum(og, axis=0, keepdims=True), float(_LCAP))
    gout_ref[11:12, :] = jnp.concatenate(
        [counts, jnp.zeros((1, _NG - _NCELLS), jnp.float32)], axis=1)

    epos = []
    ovf = []
    for s in slots:
        onehot = (s.reshape(_NG, 1) == cells_iota).astype(jnp.float32)
        pos = jnp.sum(prev * onehot, axis=1).reshape(1, _NG)
        valid = s >= 0.0
        epos.append(pos)
        ovf.append(valid & (pos >= float(_LCAP)))
    totovf = sum(o.astype(jnp.float32) for o in ovf)
    prevovf = jnp.dot(tril, totovf.reshape(_NG, 1),
                      preferred_element_type=jnp.float32).reshape(1, _NG)
    n_ovf = jnp.sum(totovf)
    within = jnp.zeros((1, _NG), jnp.float32)
    for k, s in enumerate(slots):
        is_ovf = ovf[k]
        final = jnp.where(
            s >= 0.0,
            jnp.where(is_ovf, float(_OVF_BASE) + prevovf + within,
                      s * float(_LCAP) + epos[k]),
            float(_TRASH))
        gout_ref[12 + k:13 + k, :] = s
        gout_ref[16 + k:17 + k, :] = final
        within = within + is_ovf.astype(jnp.float32)
    gout_ref[20:21, :] = jnp.full((1, _NG), n_ovf, jnp.float32)
    for r in range(21, _NGR):
        gout_ref[r:r + 1, :] = jnp.zeros((1, _NG), jnp.float32)


def _sc_kernel(pout_h, gtab_h, sem_h, out_h,
               pv, gtab_v, sem_v, list_v, out_v):
    wid = lax.axis_index("s") * 2 + lax.axis_index("c")

    pltpu.sync_copy(pout_h.at[pl.ds(wid * _NPF * _PPW, _NPF * _PPW)], pv)
    pltpu.sync_copy(gtab_h, gtab_v)
    pltpu.sync_copy(sem_h, sem_v)

    lane = lax.iota(jnp.int32, 16)

    def build(e, _):
        for s in range(4):
            ep_l = gtab_v[pl.ds((16 + s) * _NG + e * 16, 16)].astype(jnp.int32)
            gid = (e * 16 + lane).astype(jnp.float32)
            plsc.store_scatter(list_v, [ep_l], gid)
        return 0

    lax.fori_loop(0, _NG // 16, build, 0)

    n_ovf = gtab_v[pl.ds(20 * _NG, 16)].astype(jnp.int32)[0]

    def eval_block(g_l, valid, px, py, pz, tx, ty, tz, acc):
        mx = plsc.load_gather(gtab_v, [g_l])
        my = plsc.load_gather(gtab_v, [g_l + _NG])
        mz = plsc.load_gather(gtab_v, [g_l + 2 * _NG])
        jx = plsc.load_gather(gtab_v, [g_l + 3 * _NG])
        jy = plsc.load_gather(gtab_v, [g_l + 4 * _NG])
        jz = plsc.load_gather(gtab_v, [g_l + 5 * _NG])
        rr = plsc.load_gather(gtab_v, [g_l + 6 * _NG])
        ax = plsc.load_gather(gtab_v, [g_l + 7 * _NG])
        ay = plsc.load_gather(gtab_v, [g_l + 8 * _NG])
        az = plsc.load_gather(gtab_v, [g_l + 9 * _NG])
        c0 = plsc.load_gather(gtab_v, [g_l + 10 * _NG])
        dx = px - mx
        dy = py - my
        dz = pz - mz
        power = ax * (dx * dx) + ay * (dy * dy) + az * (dz * dz) + c0
        inside = ((jnp.abs(tx - jx) <= rr)
                  & (jnp.abs(ty - jy) <= rr)
                  & (jnp.abs(tz - jz) <= rr) & valid)
        w = jnp.where(inside, jnp.exp(power), 0.0)
        sbase = g_l * _NCLS
        return tuple(acc[c] + w * plsc.load_gather(sem_v, [sbase + c])
                     for c in range(_NCLS))

    def group(g, _):
        px = pv[pl.ds(0 * _PPW + g * 16, 16)]
        py = pv[pl.ds(1 * _PPW + g * 16, 16)]
        pz = pv[pl.ds(2 * _PPW + g * 16, 16)]
        tx = pv[pl.ds(3 * _PPW + g * 16, 16)]
        ty = pv[pl.ds(4 * _PPW + g * 16, 16)]
        tz = pv[pl.ds(5 * _PPW + g * 16, 16)]
        cells = pv[pl.ds(6 * _PPW + g * 16, 16)].astype(jnp.int32)
        counts = plsc.load_gather(gtab_v, [cells + 11 * _NG])
        counts_i = counts.astype(jnp.int32)
        kmax = jnp.max(counts_i)
        lbase = cells * _LCAP

        zero = jnp.zeros((16,), jnp.float32)
        acc0 = tuple(zero for _ in range(_NCLS))

        def main_body(k, acc):
            g_l = plsc.load_gather(list_v, [lbase + k]).astype(jnp.int32)
            valid = k < counts_i
            g_l = jnp.where(valid, g_l, 0)
            return eval_block(g_l, valid, px, py, pz, tx, ty, tz, acc)

        acc = lax.fori_loop(0, kmax, main_body, acc0)

        def ovf_body(k, acc):
            g_l = plsc.load_gather(
                list_v, [jnp.full((16,), _OVF_BASE, jnp.int32) + k]
            ).astype(jnp.int32)
            valid = jnp.full((16,), True)
            return eval_block(g_l, valid, px, py, pz, tx, ty, tz, acc)

        acc = lax.fori_loop(0, n_ovf, ovf_body, acc)

        rows = g * 16 + lane
        for c in range(_NCLS):
            plsc.store_scatter(out_v, [rows, jnp.full((16,), c, jnp.int32)],
                               acc[c])
        return 0

    lax.fori_loop(0, _PPW // 16, group, 0)

    pltpu.sync_copy(out_v, out_h.at[pl.ds(wid * _PPW, _PPW)])


def kernel(pts, means3D, opacities, semantics, scales, cov3D):
    p = pts[0]
    m = means3D[0].astype(jnp.float32)
    op = opacities[0].astype(jnp.float32)
    sem = semantics[0].astype(jnp.float32)
    sc = scales[0]
    cov = cov3D[0].astype(jnp.float32)

    pts_r = p.T.reshape(3, _NW, _PPW)
    cd = jnp.stack([cov[:, 0, 0], cov[:, 1, 1], cov[:, 2, 2]], axis=0)

    gout, pout = pl.pallas_call(
        _prep_kernel,
        out_shape=(jax.ShapeDtypeStruct((_NGR, _NG), jnp.float32),
                   jax.ShapeDtypeStruct((_NW, _NPF * _PPW), jnp.float32)),
    )(pts_r, m.T, sc.T, cd, op.reshape(1, _NG))

    mesh = plsc.VectorSubcoreMesh(core_axis_name="c", subcore_axis_name="s")
    fn = functools.partial(
        pl.kernel, _sc_kernel, mesh=mesh,
        compiler_params=pltpu.CompilerParams(needs_layout_passes=False),
        out_type=jax.ShapeDtypeStruct((_NPTS, _NCLS), jnp.float32),
        scratch_types=[
            pltpu.VMEM((_NPF * _PPW,), jnp.float32),
            pltpu.VMEM((_NGR * _NG,), jnp.float32),
            pltpu.VMEM((_NG * _NCLS,), jnp.float32),
            pltpu.VMEM((_LIST_LEN,), jnp.float32),
            pltpu.VMEM((_PPW, _NCLS), jnp.float32),
        ],
    )()
    out = fn(pout.reshape(-1), gout.reshape(-1), sem.reshape(-1))
    return out
